# Initial kernel scaffold; baseline (speedup 1.0000x reference)
import jax
import jax.numpy as jnp
from jax.experimental import pallas as pl


def kernel(node, edge_rel, edge_obj, edge_mask, rel_table, W0, W1):
    raise NotImplementedError("write your pallas kernel here")


# speedup vs baseline: 1.0818x; 1.0818x over previous
import functools

import jax
import jax.numpy as jnp
from jax import lax
from jax.experimental import pallas as pl
from jax.experimental.pallas import tpu as pltpu
from jax.experimental.pallas import tpu_sc as plsc

B, N, R, D, H, REL = 4, 10000, 16, 128, 128, 512
BN = B * N
NC, NS = 2, 16
NW = NC * NS
ROWS_PER_W = BN // NW
CHUNK = 5
EDGES = CHUNK * R
STEPS = ROWS_PER_W // CHUNK
NCHUNKS = BN // CHUNK
SINK = BN + REL
DV = D // 16


_mesh = plsc.VectorSubcoreMesh(
    core_axis_name="c", subcore_axis_name="s", num_cores=NC, num_subcores=NS
)


@functools.partial(
    pl.kernel,
    out_type=jax.ShapeDtypeStruct((NCHUNKS, CHUNK, D), jnp.float32),
    mesh=_mesh,
    scratch_types=[
        pltpu.VMEM((EDGES,), jnp.int32),
        pltpu.VMEM((EDGES,), jnp.int32),
        pltpu.VMEM((EDGES,), jnp.int32),
        pltpu.VMEM((EDGES,), jnp.int32),
        pltpu.VMEM((EDGES,), jnp.int32),
        pltpu.VMEM((EDGES, D), jnp.float32),
        pltpu.VMEM((EDGES, D), jnp.float32),
        pltpu.VMEM((CHUNK, D), jnp.float32),
        pltpu.SemaphoreType.DMA,
        pltpu.SemaphoreType.DMA,
    ],
)
def _sc_gather(table, obj, rel, msk, out, obj_v, rel_v, msk_v, nidx_v,
               ridx_v, nrows_v, rrows_v, stage_v, sem_n, sem_r):
    w = lax.axis_index("s") * NC + lax.axis_index("c")
    g0 = w * STEPS
    row0 = w * ROWS_PER_W
    batch_base = (row0 // N) * N

    def step(t, carry):
        g = g0 + t
        pltpu.sync_copy(obj.at[g], obj_v)
        pltpu.sync_copy(rel.at[g], rel_v)
        pltpu.sync_copy(msk.at[g], msk_v)
        for i in range(EDGES // 16):
            sl = pl.ds(i * 16, 16)
            o = obj_v[sl]
            r = rel_v[sl]
            m = msk_v[sl]
            nidx_v[sl] = jnp.where(m > 0, o + batch_base, SINK)
            ridx_v[sl] = BN + r * m
        cn = pltpu.async_copy(table.at[nidx_v], nrows_v, sem_n)
        cr = pltpu.async_copy(table.at[ridx_v], rrows_v, sem_r)
        cn.wait()
        cr.wait()
        for c in range(CHUNK):
            def ebody(e, acc):
                row = c * R + e
                return tuple(
                    acc[d]
                    + nrows_v[row, pl.ds(d * 16, 16)]
                    + rrows_v[row, pl.ds(d * 16, 16)]
                    for d in range(DV)
                )
            acc = lax.fori_loop(
                0, R, ebody,
                tuple(jnp.zeros((16,), jnp.float32) for _ in range(DV)),
            )
            for d in range(DV):
                stage_v[c, pl.ds(d * 16, 16)] = acc[d]
        pltpu.sync_copy(stage_v, out.at[g])
        return carry

    lax.fori_loop(0, STEPS, step, 0)


RB = 400


def _tc_body(s_ref, node_ref, mask_ref, w0_ref, w1_ref, out_ref):
    m = mask_ref[...].astype(jnp.float32)
    elen = jnp.maximum(jnp.sum(m, axis=1, keepdims=True), 1.0)
    eh = s_ref[...] * (1.0 / (elen * elen))
    sh = lax.dot_general(node_ref[...], w0_ref[...],
                         (((1,), (1,)), ((), ())),
                         preferred_element_type=jnp.float32)
    eh = lax.dot_general(eh, w1_ref[...],
                         (((1,), (1,)), ((), ())),
                         preferred_element_type=jnp.float32)
    out_ref[...] = jnp.maximum(sh + eh, 0.0)


_tc_finish = pl.pallas_call(
    _tc_body,
    grid=(BN // RB,),
    in_specs=[
        pl.BlockSpec((RB, D), lambda i: (i, 0)),
        pl.BlockSpec((RB, D), lambda i: (i, 0)),
        pl.BlockSpec((RB, R), lambda i: (i, 0)),
        pl.BlockSpec((H, D), lambda i: (0, 0)),
        pl.BlockSpec((H, D), lambda i: (0, 0)),
    ],
    out_specs=pl.BlockSpec((RB, H), lambda i: (i, 0)),
    out_shape=jax.ShapeDtypeStruct((BN, H), jnp.float32),
)


def kernel(node, edge_rel, edge_obj, edge_mask, rel_table, W0, W1):
    node_flat = node.reshape(BN, D)
    table = jnp.concatenate(
        [node_flat, -rel_table, jnp.zeros((1, D), jnp.float32)], axis=0)
    obj2 = edge_obj.reshape(NCHUNKS, EDGES).astype(jnp.int32)
    rel2 = edge_rel.reshape(NCHUNKS, EDGES).astype(jnp.int32)
    msk2 = edge_mask.reshape(NCHUNKS, EDGES).astype(jnp.int32)
    s = _sc_gather(table, obj2, rel2, msk2).reshape(BN, D)
    out = _tc_finish(s, node_flat, edge_mask.reshape(BN, R).astype(jnp.int32),
                     W0, W1)
    return out.reshape(B, N, H)


# speedup vs baseline: 11.8653x; 10.9679x over previous
import functools

import jax
import jax.numpy as jnp
from jax import lax
from jax.experimental import pallas as pl
from jax.experimental.pallas import tpu as pltpu
from jax.experimental.pallas import tpu_sc as plsc

B, N, R, D, H, REL = 4, 10000, 16, 128, 128, 512
BN = B * N
NC, NS = 2, 16
NW = NC * NS
ROWS_PER_W = BN // NW
CHUNK = 5
EDGES = CHUNK * R
STEPS = ROWS_PER_W // CHUNK
NCHUNKS = BN // CHUNK
SINKB = BN + REL
NSINK = 256
DV = D // 16


_mesh = plsc.VectorSubcoreMesh(
    core_axis_name="c", subcore_axis_name="s", num_cores=NC, num_subcores=NS
)


@functools.partial(
    pl.kernel,
    out_type=jax.ShapeDtypeStruct((NCHUNKS, CHUNK, D), jnp.float32),
    mesh=_mesh,
    scratch_types=[
        pltpu.VMEM((EDGES,), jnp.int32),
        pltpu.VMEM((EDGES,), jnp.int32),
        pltpu.VMEM((EDGES,), jnp.int32),
        pltpu.VMEM((EDGES,), jnp.int32),
        pltpu.VMEM((EDGES,), jnp.int32),
        pltpu.VMEM((EDGES, D), jnp.float32),
        pltpu.VMEM((EDGES, D), jnp.float32),
        pltpu.VMEM((CHUNK, D), jnp.float32),
        pltpu.SemaphoreType.DMA,
        pltpu.SemaphoreType.DMA,
    ],
)
def _sc_gather(table, obj, rel, msk, out, obj_v, rel_v, msk_v, nidx_v,
               ridx_v, nrows_v, rrows_v, stage_v, sem_n, sem_r):
    w = lax.axis_index("s") * NC + lax.axis_index("c")
    g0 = w * STEPS
    row0 = w * ROWS_PER_W
    batch_base = (row0 // N) * N

    def step(t, carry):
        g = g0 + t
        pltpu.sync_copy(obj.at[g], obj_v)
        pltpu.sync_copy(rel.at[g], rel_v)
        pltpu.sync_copy(msk.at[g], msk_v)
        for i in range(EDGES // 16):
            sl = pl.ds(i * 16, 16)
            o = obj_v[sl]
            r = rel_v[sl]
            m = msk_v[sl]
            keep = m > 0
            nidx_v[sl] = jnp.where(keep, o + batch_base,
                                   SINKB + (o & (NSINK - 1)))
            ridx_v[sl] = jnp.where(keep, BN + r,
                                   SINKB + (r & (NSINK - 1)))
        cn = pltpu.async_copy(table.at[nidx_v], nrows_v, sem_n)
        cr = pltpu.async_copy(table.at[ridx_v], rrows_v, sem_r)
        cn.wait()
        cr.wait()
        for c in range(CHUNK):
            def ebody(e, acc):
                row = c * R + e
                return tuple(
                    acc[d]
                    + nrows_v[row, pl.ds(d * 16, 16)]
                    + rrows_v[row, pl.ds(d * 16, 16)]
                    for d in range(DV)
                )
            acc = lax.fori_loop(
                0, R, ebody,
                tuple(jnp.zeros((16,), jnp.float32) for _ in range(DV)),
            )
            for d in range(DV):
                stage_v[c, pl.ds(d * 16, 16)] = acc[d]
        pltpu.sync_copy(stage_v, out.at[g])
        return carry

    lax.fori_loop(0, STEPS, step, 0)


RB = 400


def _tc_body(s_ref, node_ref, mask_ref, w0_ref, w1_ref, out_ref):
    m = mask_ref[...].astype(jnp.float32)
    elen = jnp.maximum(jnp.sum(m, axis=1, keepdims=True), 1.0)
    eh = s_ref[...] * (1.0 / (elen * elen))
    sh = lax.dot_general(node_ref[...], w0_ref[...],
                         (((1,), (1,)), ((), ())),
                         preferred_element_type=jnp.float32)
    eh = lax.dot_general(eh, w1_ref[...],
                         (((1,), (1,)), ((), ())),
                         preferred_element_type=jnp.float32)
    out_ref[...] = jnp.maximum(sh + eh, 0.0)


_tc_finish = pl.pallas_call(
    _tc_body,
    grid=(BN // RB,),
    in_specs=[
        pl.BlockSpec((RB, D), lambda i: (i, 0)),
        pl.BlockSpec((RB, D), lambda i: (i, 0)),
        pl.BlockSpec((RB, R), lambda i: (i, 0)),
        pl.BlockSpec((H, D), lambda i: (0, 0)),
        pl.BlockSpec((H, D), lambda i: (0, 0)),
    ],
    out_specs=pl.BlockSpec((RB, H), lambda i: (i, 0)),
    out_shape=jax.ShapeDtypeStruct((BN, H), jnp.float32),
)


def kernel(node, edge_rel, edge_obj, edge_mask, rel_table, W0, W1):
    node_flat = node.reshape(BN, D)
    table = jnp.concatenate(
        [node_flat, -rel_table, jnp.zeros((NSINK, D), jnp.float32)], axis=0)
    obj2 = edge_obj.reshape(NCHUNKS, EDGES).astype(jnp.int32)
    rel2 = edge_rel.reshape(NCHUNKS, EDGES).astype(jnp.int32)
    msk2 = edge_mask.reshape(NCHUNKS, EDGES).astype(jnp.int32)
    s = _sc_gather(table, obj2, rel2, msk2).reshape(BN, D)
    out = _tc_finish(s, node_flat, edge_mask.reshape(BN, R).astype(jnp.int32),
                     W0, W1)
    return out.reshape(B, N, H)


# speedup vs baseline: 19.1852x; 1.6169x over previous
import functools

import jax
import jax.numpy as jnp
from jax import lax
from jax.experimental import pallas as pl
from jax.experimental.pallas import tpu as pltpu
from jax.experimental.pallas import tpu_sc as plsc

B, N, R, D, H, REL = 4, 10000, 16, 128, 128, 512
BN = B * N
NC, NS = 2, 16
NW = NC * NS
ROWS_PER_W = BN // NW
CHUNK = 5
EDGES = CHUNK * R
STEPS = ROWS_PER_W // CHUNK
NCHUNKS = BN // CHUNK
SINKB = BN + REL
NSINK = 256
DV = D // 16


_mesh = plsc.VectorSubcoreMesh(
    core_axis_name="c", subcore_axis_name="s", num_cores=NC, num_subcores=NS
)


@functools.partial(
    pl.kernel,
    out_type=jax.ShapeDtypeStruct((NCHUNKS, CHUNK, D), jnp.float32),
    mesh=_mesh,
    scratch_types=[
        pltpu.VMEM((3, EDGES), jnp.int32),
        pltpu.VMEM((3, EDGES), jnp.int32),
        pltpu.VMEM((EDGES,), jnp.int32),
        pltpu.VMEM((EDGES,), jnp.int32),
        pltpu.VMEM((EDGES,), jnp.int32),
        pltpu.VMEM((EDGES,), jnp.int32),
        pltpu.VMEM((EDGES, D), jnp.float32),
        pltpu.VMEM((EDGES, D), jnp.float32),
        pltpu.VMEM((EDGES, D), jnp.float32),
        pltpu.VMEM((EDGES, D), jnp.float32),
        pltpu.VMEM((CHUNK, D), jnp.float32),
        pltpu.VMEM((CHUNK, D), jnp.float32),
        pltpu.SemaphoreType.DMA,
        pltpu.SemaphoreType.DMA,
        pltpu.SemaphoreType.DMA,
        pltpu.SemaphoreType.DMA,
        pltpu.SemaphoreType.DMA,
        pltpu.SemaphoreType.DMA,
        pltpu.SemaphoreType.DMA,
        pltpu.SemaphoreType.DMA,
    ],
)
def _sc_gather(table, idx_in, out,
               ib0, ib1, ni0, ni1, ri0, ri1, nr0, nr1, rr0, rr1, st0, st1,
               si0, si1, sgn0, sgn1, sgr0, sgr1, so0, so1):
    ib = (ib0, ib1)
    ni = (ni0, ni1)
    ri = (ri0, ri1)
    nr = (nr0, nr1)
    rr = (rr0, rr1)
    st = (st0, st1)
    si = (si0, si1)
    sgn = (sgn0, sgn1)
    sgr = (sgr0, sgr1)
    so = (so0, so1)

    w = lax.axis_index("s") * NC + lax.axis_index("c")
    g0 = w * STEPS
    row0 = w * ROWS_PER_W
    batch_base = (row0 // N) * N

    def gidx_and_fire(t, p):
        for i in range(EDGES // 16):
            sl = pl.ds(i * 16, 16)
            o = ib[p][0, sl]
            r = ib[p][1, sl]
            m = ib[p][2, sl]
            keep = m > 0
            ni[p][sl] = jnp.where(keep, o + batch_base,
                                  SINKB + (o & (NSINK - 1)))
            ri[p][sl] = jnp.where(keep, BN + r, SINKB + (r & (NSINK - 1)))
        pltpu.async_copy(table.at[ni[p]], nr[p], sgn[p])
        pltpu.async_copy(table.at[ri[p]], rr[p], sgr[p])

    def load_idx(t, p):
        pltpu.async_copy(idx_in.at[g0 + t], ib[p], si[p])

    def drain(dummy_src, dst, sem):
        pltpu.make_async_copy(dummy_src, dst, sem).wait()

    load_idx(0, 0)
    load_idx(1, 1)
    drain(idx_in.at[0], ib[0], si[0])
    gidx_and_fire(0, 0)

    def pair(k, carry):
        for par in (0, 1):
            t = 2 * k + par
            q = 1 - par
            if par == 0:
                drain(idx_in.at[0], ib[q], si[q])
                gidx_and_fire(t + 1, q)
            else:
                @pl.when(k < STEPS // 2 - 1)
                def _():
                    drain(idx_in.at[0], ib[q], si[q])
                    gidx_and_fire(t + 1, q)
            @pl.when(k < STEPS // 2 - 1)
            def _():
                load_idx(t + 2, par)
            drain(table.at[pl.ds(0, EDGES)], nr[par], sgn[par])
            drain(table.at[pl.ds(0, EDGES)], rr[par], sgr[par])
            @pl.when(k >= 1)
            def _():
                drain(st[par], out.at[0], so[par])
            for c in range(CHUNK):
                def ebody(e, acc):
                    row = c * R + e
                    return tuple(
                        acc[d]
                        + nr[par][row, pl.ds(d * 16, 16)]
                        + rr[par][row, pl.ds(d * 16, 16)]
                        for d in range(DV)
                    )
                acc = lax.fori_loop(
                    0, R, ebody,
                    tuple(jnp.zeros((16,), jnp.float32) for _ in range(DV)),
                )
                for d in range(DV):
                    st[par][c, pl.ds(d * 16, 16)] = acc[d]
            pltpu.async_copy(st[par], out.at[g0 + t], so[par])
        return carry

    lax.fori_loop(0, STEPS // 2, pair, 0)
    drain(st[0], out.at[0], so[0])
    drain(st[1], out.at[0], so[1])


RB = 400


def _tc_body(s_ref, node_ref, mask_ref, w0_ref, w1_ref, out_ref):
    m = mask_ref[...].astype(jnp.float32)
    elen = jnp.maximum(jnp.sum(m, axis=1, keepdims=True), 1.0)
    eh = s_ref[...] * (1.0 / (elen * elen))
    sh = lax.dot_general(node_ref[...], w0_ref[...],
                         (((1,), (1,)), ((), ())),
                         preferred_element_type=jnp.float32)
    eh = lax.dot_general(eh, w1_ref[...],
                         (((1,), (1,)), ((), ())),
                         preferred_element_type=jnp.float32)
    out_ref[...] = jnp.maximum(sh + eh, 0.0)


_tc_finish = pl.pallas_call(
    _tc_body,
    grid=(BN // RB,),
    in_specs=[
        pl.BlockSpec((RB, D), lambda i: (i, 0)),
        pl.BlockSpec((RB, D), lambda i: (i, 0)),
        pl.BlockSpec((RB, R), lambda i: (i, 0)),
        pl.BlockSpec((H, D), lambda i: (0, 0)),
        pl.BlockSpec((H, D), lambda i: (0, 0)),
    ],
    out_specs=pl.BlockSpec((RB, H), lambda i: (i, 0)),
    out_shape=jax.ShapeDtypeStruct((BN, H), jnp.float32),
)


def kernel(node, edge_rel, edge_obj, edge_mask, rel_table, W0, W1):
    node_flat = node.reshape(BN, D)
    table = jnp.concatenate(
        [node_flat, -rel_table, jnp.zeros((NSINK, D), jnp.float32)], axis=0)
    idx_in = jnp.stack(
        [edge_obj.reshape(NCHUNKS, EDGES).astype(jnp.int32),
         edge_rel.reshape(NCHUNKS, EDGES).astype(jnp.int32),
         edge_mask.reshape(NCHUNKS, EDGES).astype(jnp.int32)], axis=1)
    s = _sc_gather(table, idx_in).reshape(BN, D)
    out = _tc_finish(s, node_flat, edge_mask.reshape(BN, R).astype(jnp.int32),
                     W0, W1)
    return out.reshape(B, N, H)
